# baseline (speedup 1.0000x reference)
import jax
import jax.numpy as jnp
from jax.experimental import pallas as pl
from jax.experimental.pallas import tpu as pltpu

_TB = 16384
_EAR_EMB_DIM = 32


def _round_up(x, m):
    return (x + m - 1) // m * m


def _linear_block_kernel(x_ref, w_ref, b_ref, o_ref):
    acc = jnp.dot(x_ref[...], w_ref[...], preferred_element_type=jnp.float32)
    o_ref[...] = (acc + b_ref[...]).astype(o_ref.dtype)


def kernel(ear_anthro, weight_t, bias):
    B, d_in = ear_anthro.shape
    d_out = _EAR_EMB_DIM

    w = weight_t[:d_in, :d_out]
    b2 = bias[:d_out].reshape(1, d_out)
    x = ear_anthro.astype(jnp.float32)

    tb = min(_TB, _round_up(B, 8))
    b_grid = _round_up(B, tb)
    if b_grid != B:
        x = jnp.pad(x, ((0, b_grid - B), (0, 0)))

    out = pl.pallas_call(
        _linear_block_kernel,
        out_shape=jax.ShapeDtypeStruct((b_grid, d_out), jnp.float32),
        grid_spec=pl.GridSpec(
            grid=(b_grid // tb,),
            in_specs=[
                pl.BlockSpec((tb, d_in), lambda i: (i, 0)),
                pl.BlockSpec((d_in, d_out), lambda i: (0, 0)),
                pl.BlockSpec((1, d_out), lambda i: (0, 0)),
            ],
            out_specs=pl.BlockSpec((tb, d_out), lambda i: (i, 0)),
        ),
        compiler_params=pltpu.CompilerParams(
            dimension_semantics=("parallel",)),
        cost_estimate=pl.CostEstimate(
            flops=2 * b_grid * d_in * d_out,
            transcendentals=0,
            bytes_accessed=4 * (b_grid * d_in + d_in * d_out
                                + d_out + b_grid * d_out)),
    )(x, w, b2)

    if b_grid != B:
        out = out[:B]
    return out
